# baseline (speedup 1.0000x reference)
import functools

import jax
import jax.numpy as jnp
from jax import lax
from jax.experimental import pallas as pl
from jax.experimental.pallas import tpu as pltpu
from jax.experimental.pallas import tpu_sc as plsc

B, P, CP, C0, CH, K = 8, 2048, 3, 64, 64, 16
N = B * P
E = N * K

BLKP = 1024
BLK = 256
BLKC = 512

NW = 32
CHUNK = 128
G = 2
BG = B // G
NG = BG * P
EG_ = NG * K

_INF = float("inf")


def _pre_body(f_ref, wt_ref, wc_ref, wsc_ref, b0_ref, bsc_ref,
              t_ref, c_ref, s_ref):
    f = f_ref[...]
    t_ref[...] = jnp.dot(f, wt_ref[...], preferred_element_type=jnp.float32)
    c_ref[...] = jnp.dot(f, wc_ref[...], preferred_element_type=jnp.float32) + b0_ref[...]
    s_ref[...] = jnp.dot(f, wsc_ref[...], preferred_element_type=jnp.float32) + bsc_ref[...]


def _knn_body(pts_ref, ptsT_ref, idx_ref, *, bbase):
    b = bbase + pl.program_id(0)
    i = pl.program_id(1)
    q = pts_ref[0]
    pt = ptsT_ref[0]
    qx = q[:, 0:1]
    qy = q[:, 1:2]
    qz = q[:, 2:3]
    px = pt[0:1, :]
    py = pt[1:2, :]
    pz = pt[2:3, :]
    m = jnp.dot(q, pt, preferred_element_type=jnp.float32)
    rq = qx * qx + qy * qy + qz * qz
    rp = px * px + py * py + pz * pz
    d = (rp - 2.0 * m) + rq
    cand = lax.broadcasted_iota(jnp.int32, (P, BLK), 0)
    xi = lax.bitcast_convert_type(d, jnp.int32)
    s = xi ^ ((xi >> 31) & jnp.int32(0x7FFFFFFF))
    mx = jnp.int32(0x7FFFFFFF)
    key = (s & jnp.int32(~2047)) | cand
    rowid = i * BLK + lax.broadcasted_iota(jnp.int32, (P, BLK), 1)
    selfmask = cand == rowid
    key = jnp.where(selfmask, mx, key)
    mdiag = jnp.sum(jnp.where(selfmask, m, 0.0), axis=0, keepdims=True)
    dself = (rp - 2.0 * mdiag) + rp
    sxi = lax.bitcast_convert_type(dself, jnp.int32)
    sss = sxi ^ ((sxi >> 31) & jnp.int32(0x7FFFFFFF))
    selfidx = i * BLK + lax.broadcasted_iota(jnp.int32, (1, BLK), 1)
    skey = (sss & jnp.int32(~2047)) | selfidx

    def ce(S, a, b):
        lt = S[b] < S[a]
        S[a], S[b] = jnp.where(lt, S[b], S[a]), jnp.where(lt, S[a], S[b])

    S = [key]
    n = P
    while n > 1:
        h = n // 2
        A = [t[:h] for t in S]
        Brev = [t[h:] for t in reversed(S)]
        k = len(S)
        if k < K:
            S = A + Brev
            k2, dist = 2 * k, k
        else:
            S = [jnp.minimum(a, b) for a, b in zip(A, Brev)]
            k2, dist = k, k // 2
        dstep = dist
        while dstep >= 1:
            for i0 in range(0, k2, 2 * dstep):
                for ii in range(i0, i0 + dstep):
                    ce(S, ii, ii + dstep)
            dstep //= 2
        n = h

    outs = [t & 2047 for t in S]
    outs[0] = jnp.where(skey < S[0], outs[0], selfidx)
    idx_ref[...] = jnp.concatenate(outs, axis=0) + b * P


def _sc_gather(t_tab, idx_flat):
    EG = idx_flat.shape[0]
    EW = EG // NW
    NCH = EW // CHUNK
    mesh = plsc.VectorSubcoreMesh(core_axis_name="c", subcore_axis_name="s")

    @functools.partial(
        pl.kernel,
        out_type=jax.ShapeDtypeStruct((EG, C0), jnp.float32),
        mesh=mesh,
        scratch_types=[
            pltpu.VMEM((2, CHUNK), jnp.int32),
            pltpu.VMEM((CHUNK, C0), jnp.float32),
            pltpu.VMEM((CHUNK, C0), jnp.float32),
            pltpu.SemaphoreType.DMA,
            pltpu.SemaphoreType.DMA,
        ],
        compiler_params=pltpu.CompilerParams(use_tc_tiling_on_sc=False),
    )
    def gather_kernel(t_hbm, idx_hbm, out_hbm, idx_v, rows0, rows1, sem0, sem1):
        wid = lax.axis_index("s") * 2 + lax.axis_index("c")
        base = wid * EW
        rows = (rows0, rows1)
        sems = (sem0, sem1)

        pltpu.sync_copy(idx_hbm.at[pl.ds(base, CHUNK)], idx_v.at[0])
        pltpu.async_copy(t_hbm.at[idx_v.at[0]], rows0, sem0)

        def pair(c2, carry):
            c = c2 * 2
            for j in range(2):
                cc = c + j
                nb = (j + 1) % 2
                @pl.when(cc + 1 < NCH)
                def _():
                    nxt = base + (cc + 1) * CHUNK
                    pltpu.sync_copy(idx_hbm.at[pl.ds(nxt, CHUNK)], idx_v.at[nb])
                    pltpu.async_copy(t_hbm.at[idx_v.at[nb]], rows[nb], sems[nb])
                pltpu.make_async_copy(t_hbm.at[idx_v.at[j]], rows[j], sems[j]).wait()
                pltpu.sync_copy(rows[j], out_hbm.at[pl.ds(base + cc * CHUNK, CHUNK)])
            return carry

        lax.fori_loop(0, NCH // 2, pair, 0)

    return gather_kernel(t_tab, idx_flat)


def _mlp_body(tg_ref, c_ref, s_ref, w1_ref, b1_ref, w2_ref, b2_ref, o_ref):
    c = c_ref[...]
    w1 = w1_ref[...]
    w2 = w2_ref[...]
    b1 = b1_ref[...]
    b2 = b2_ref[...]
    acc = jnp.zeros((BLKC, CH), jnp.float32)
    for k in range(K):
        y = jnp.maximum(c + tg_ref[k], 0.0)
        y = jnp.maximum(jnp.dot(y, w1, preferred_element_type=jnp.float32) + b1, 0.0)
        y = jnp.maximum(jnp.dot(y, w2, preferred_element_type=jnp.float32) + b2, 0.0)
        acc = acc + y
    o_ref[...] = jnp.maximum(s_ref[...] + acc * (1.0 / K), 0.0)


def kernel(points, features, W0, W1, W2, Wsc, g0, b0, g1, b1, g2, b2, gsc, bsc):
    inv = 1.0 / jnp.sqrt(jnp.float32(1.0) + jnp.float32(1e-3))
    s0 = g0 * inv
    s1 = g1 * inv
    s2 = g2 * inv
    ssc = gsc * inv
    W0a = W0[:C0] * s0[None, :]
    W0b = W0[C0:] * s0[None, :]
    Wc = W0a - W0b
    W1s = W1 * s1[None, :]
    W2s = W2 * s2[None, :]
    Wscs = Wsc * ssc[None, :]

    f2 = features.reshape(N, C0)
    wspec = pl.BlockSpec((C0, C0), lambda i: (0, 0))
    bspec = pl.BlockSpec((1, C0), lambda i: (0, 0))
    rspec = pl.BlockSpec((BLKP, C0), lambda i: (i, 0))
    t, c0m, sc0 = pl.pallas_call(
        _pre_body,
        grid=(N // BLKP,),
        in_specs=[rspec, wspec, wspec, wspec, bspec, bspec],
        out_specs=[rspec, rspec, rspec],
        out_shape=[jax.ShapeDtypeStruct((N, C0), jnp.float32)] * 3,
    )(f2, W0b, Wc, Wscs, b0.reshape(1, C0), bsc.reshape(1, C0))

    ptsT = jnp.swapaxes(points, 1, 2)
    b1r = b1.reshape(1, C0)
    b2r = b2.reshape(1, C0)

    idxs = []
    for g in range(G):
        idxs.append(pl.pallas_call(
            functools.partial(_knn_body, bbase=g * BG),
            grid=(BG, P // BLK),
            in_specs=[
                pl.BlockSpec((1, P, CP), lambda b, i: (b, 0, 0)),
                pl.BlockSpec((1, CP, BLK), lambda b, i: (b, 0, i)),
            ],
            out_specs=pl.BlockSpec((K, BLK), lambda b, i: (0, b * (P // BLK) + i)),
            out_shape=jax.ShapeDtypeStruct((K, NG), jnp.int32),
        )(points[g * BG:(g + 1) * BG], ptsT[g * BG:(g + 1) * BG]))

    outs = []
    for g in range(G):
        tg3 = _sc_gather(t, idxs[g].reshape(EG_)).reshape(K, NG, C0)
        outs.append(pl.pallas_call(
            _mlp_body,
            grid=(NG // BLKC,),
            in_specs=[
                pl.BlockSpec((K, BLKC, C0), lambda i: (0, i, 0)),
                pl.BlockSpec((BLKC, C0), lambda i, g=g: (g * (NG // BLKC) + i, 0)),
                pl.BlockSpec((BLKC, C0), lambda i, g=g: (g * (NG // BLKC) + i, 0)),
                wspec, bspec, wspec, bspec,
            ],
            out_specs=pl.BlockSpec((BLKC, C0), lambda i: (i, 0)),
            out_shape=jax.ShapeDtypeStruct((NG, C0), jnp.float32),
        )(tg3, c0m, sc0, W1s, b1r, W2s, b2r))

    return jnp.concatenate(outs, axis=0).reshape(B, P, CH)


# speedup vs baseline: 1.1996x; 1.1996x over previous
import functools

import jax
import jax.numpy as jnp
from jax import lax
from jax.experimental import pallas as pl
from jax.experimental.pallas import tpu as pltpu
from jax.experimental.pallas import tpu_sc as plsc

B, P, CP, C0, CH, K = 8, 2048, 3, 64, 64, 16
N = B * P
E = N * K

BLKP = 1024
BLK = 256
BLKC = 512

NW = 32
CHUNK = 128
G = 2
BG = B // G
NG = BG * P
EG_ = NG * K

_INF = float("inf")


def _pre_body(f_ref, wt_ref, wc_ref, wsc_ref, b0_ref, bsc_ref,
              t_ref, c_ref, s_ref):
    f = f_ref[...]
    t_ref[...] = jnp.dot(f, wt_ref[...], preferred_element_type=jnp.float32)
    c_ref[...] = jnp.dot(f, wc_ref[...], preferred_element_type=jnp.float32) + b0_ref[...]
    s_ref[...] = jnp.dot(f, wsc_ref[...], preferred_element_type=jnp.float32) + bsc_ref[...]


def _knn_body(pts_ref, ptsT_ref, idx_ref, *, bbase):
    b = bbase + pl.program_id(0)
    i = pl.program_id(1)
    q = pts_ref[0]
    pt = ptsT_ref[0]
    qx = q[:, 0:1]
    qy = q[:, 1:2]
    qz = q[:, 2:3]
    px = pt[0:1, :]
    py = pt[1:2, :]
    pz = pt[2:3, :]
    m = jnp.dot(q, pt, preferred_element_type=jnp.float32)
    rq = qx * qx + qy * qy + qz * qz
    rp = px * px + py * py + pz * pz
    d = (rp - 2.0 * m) + rq
    cand = lax.broadcasted_iota(jnp.int32, (P, BLK), 0)
    xi = lax.bitcast_convert_type(d, jnp.int32)
    key = lax.bitcast_convert_type((xi & jnp.int32(~2047)) | cand, jnp.float32)
    rowid = i * BLK + lax.broadcasted_iota(jnp.int32, (P, BLK), 1)
    selfmask = cand == rowid
    key = jnp.where(selfmask, _INF, key)
    mdiag = jnp.sum(jnp.where(selfmask, m, 0.0), axis=0, keepdims=True)
    dself = (rp - 2.0 * mdiag) + rp
    sxi = lax.bitcast_convert_type(dself, jnp.int32)
    selfidx = i * BLK + lax.broadcasted_iota(jnp.int32, (1, BLK), 1)
    skey = lax.bitcast_convert_type((sxi & jnp.int32(~2047)) | selfidx,
                                    jnp.float32)

    def ce(S, a, b):
        S[a], S[b] = jnp.minimum(S[a], S[b]), jnp.maximum(S[a], S[b])

    S = [key]
    n = P
    while n > 1:
        h = n // 2
        A = [t[:h] for t in S]
        Brev = [t[h:] for t in reversed(S)]
        k = len(S)
        if k < K:
            S = A + Brev
            k2, dist = 2 * k, k
        else:
            S = [jnp.minimum(a, b) for a, b in zip(A, Brev)]
            k2, dist = k, k // 2
        dstep = dist
        while dstep >= 1:
            for i0 in range(0, k2, 2 * dstep):
                for ii in range(i0, i0 + dstep):
                    ce(S, ii, ii + dstep)
            dstep //= 2
        n = h

    outs = [lax.bitcast_convert_type(t, jnp.int32) & 2047 for t in S]
    outs[0] = jnp.where(skey < S[0], outs[0], selfidx)
    idx_ref[...] = jnp.concatenate(outs, axis=0) + b * P


def _sc_gather(t_tab, idx_flat):
    EG = idx_flat.shape[0]
    EW = EG // NW
    NCH = EW // CHUNK
    mesh = plsc.VectorSubcoreMesh(core_axis_name="c", subcore_axis_name="s")

    @functools.partial(
        pl.kernel,
        out_type=jax.ShapeDtypeStruct((EG, C0), jnp.float32),
        mesh=mesh,
        scratch_types=[
            pltpu.VMEM((2, CHUNK), jnp.int32),
            pltpu.VMEM((CHUNK, C0), jnp.float32),
            pltpu.VMEM((CHUNK, C0), jnp.float32),
            pltpu.SemaphoreType.DMA,
            pltpu.SemaphoreType.DMA,
        ],
        compiler_params=pltpu.CompilerParams(use_tc_tiling_on_sc=False),
    )
    def gather_kernel(t_hbm, idx_hbm, out_hbm, idx_v, rows0, rows1, sem0, sem1):
        wid = lax.axis_index("s") * 2 + lax.axis_index("c")
        base = wid * EW
        rows = (rows0, rows1)
        sems = (sem0, sem1)

        pltpu.sync_copy(idx_hbm.at[pl.ds(base, CHUNK)], idx_v.at[0])
        pltpu.async_copy(t_hbm.at[idx_v.at[0]], rows0, sem0)

        def pair(c2, carry):
            c = c2 * 2
            for j in range(2):
                cc = c + j
                nb = (j + 1) % 2
                @pl.when(cc + 1 < NCH)
                def _():
                    nxt = base + (cc + 1) * CHUNK
                    pltpu.sync_copy(idx_hbm.at[pl.ds(nxt, CHUNK)], idx_v.at[nb])
                    pltpu.async_copy(t_hbm.at[idx_v.at[nb]], rows[nb], sems[nb])
                pltpu.make_async_copy(t_hbm.at[idx_v.at[j]], rows[j], sems[j]).wait()
                pltpu.sync_copy(rows[j], out_hbm.at[pl.ds(base + cc * CHUNK, CHUNK)])
            return carry

        lax.fori_loop(0, NCH // 2, pair, 0)

    return gather_kernel(t_tab, idx_flat)


def _mlp_body(tg_ref, c_ref, s_ref, w1_ref, b1_ref, w2_ref, b2_ref, o_ref):
    c = c_ref[...]
    w1 = w1_ref[...]
    w2 = w2_ref[...]
    b1 = b1_ref[...]
    b2 = b2_ref[...]
    acc = jnp.zeros((BLKC, CH), jnp.float32)
    for k in range(K):
        y = jnp.maximum(c + tg_ref[k], 0.0)
        y = jnp.maximum(jnp.dot(y, w1, preferred_element_type=jnp.float32) + b1, 0.0)
        y = jnp.maximum(jnp.dot(y, w2, preferred_element_type=jnp.float32) + b2, 0.0)
        acc = acc + y
    o_ref[...] = jnp.maximum(s_ref[...] + acc * (1.0 / K), 0.0)


def kernel(points, features, W0, W1, W2, Wsc, g0, b0, g1, b1, g2, b2, gsc, bsc):
    inv = 1.0 / jnp.sqrt(jnp.float32(1.0) + jnp.float32(1e-3))
    s0 = g0 * inv
    s1 = g1 * inv
    s2 = g2 * inv
    ssc = gsc * inv
    W0a = W0[:C0] * s0[None, :]
    W0b = W0[C0:] * s0[None, :]
    Wc = W0a - W0b
    W1s = W1 * s1[None, :]
    W2s = W2 * s2[None, :]
    Wscs = Wsc * ssc[None, :]

    f2 = features.reshape(N, C0)
    wspec = pl.BlockSpec((C0, C0), lambda i: (0, 0))
    bspec = pl.BlockSpec((1, C0), lambda i: (0, 0))
    rspec = pl.BlockSpec((BLKP, C0), lambda i: (i, 0))
    t, c0m, sc0 = pl.pallas_call(
        _pre_body,
        grid=(N // BLKP,),
        in_specs=[rspec, wspec, wspec, wspec, bspec, bspec],
        out_specs=[rspec, rspec, rspec],
        out_shape=[jax.ShapeDtypeStruct((N, C0), jnp.float32)] * 3,
    )(f2, W0b, Wc, Wscs, b0.reshape(1, C0), bsc.reshape(1, C0))

    ptsT = jnp.swapaxes(points, 1, 2)
    b1r = b1.reshape(1, C0)
    b2r = b2.reshape(1, C0)

    idxs = []
    for g in range(G):
        idxs.append(pl.pallas_call(
            functools.partial(_knn_body, bbase=g * BG),
            grid=(BG, P // BLK),
            in_specs=[
                pl.BlockSpec((1, P, CP), lambda b, i: (b, 0, 0)),
                pl.BlockSpec((1, CP, BLK), lambda b, i: (b, 0, i)),
            ],
            out_specs=pl.BlockSpec((K, BLK), lambda b, i: (0, b * (P // BLK) + i)),
            out_shape=jax.ShapeDtypeStruct((K, NG), jnp.int32),
        )(points[g * BG:(g + 1) * BG], ptsT[g * BG:(g + 1) * BG]))

    outs = []
    for g in range(G):
        tg3 = _sc_gather(t, idxs[g].reshape(EG_)).reshape(K, NG, C0)
        outs.append(pl.pallas_call(
            _mlp_body,
            grid=(NG // BLKC,),
            in_specs=[
                pl.BlockSpec((K, BLKC, C0), lambda i: (0, i, 0)),
                pl.BlockSpec((BLKC, C0), lambda i, g=g: (g * (NG // BLKC) + i, 0)),
                pl.BlockSpec((BLKC, C0), lambda i, g=g: (g * (NG // BLKC) + i, 0)),
                wspec, bspec, wspec, bspec,
            ],
            out_specs=pl.BlockSpec((BLKC, C0), lambda i: (i, 0)),
            out_shape=jax.ShapeDtypeStruct((NG, C0), jnp.float32),
        )(tg3, c0m, sc0, W1s, b1r, W2s, b2r))

    return jnp.concatenate(outs, axis=0).reshape(B, P, CH)
